# Initial kernel scaffold; baseline (speedup 1.0000x reference)
import jax
import jax.numpy as jnp
from jax.experimental import pallas as pl


def kernel(x, edge_index, edge_attr, batch, Wn_o, bn_o, We_o, be_o, Wm_o, a_o, Wn_f, bn_f, We_f, be_f, Wm_f, a_f, Wn_j, bn_j, We_j, be_j, Wm_j, a_j, Wo_o, bo_o, Wo_f, bo_f, Wp1, bp1, Wp2a, bp2a, Wp2b, bp2b):
    raise NotImplementedError("write your pallas kernel here")


# speedup vs baseline: 4.5074x; 4.5074x over previous
import functools

import jax
import jax.numpy as jnp
from jax import lax
from jax.experimental import pallas as pl
from jax.experimental.pallas import tpu as pltpu
from jax.experimental.pallas import tpu_sc as plsc

N = 10000
E = 320000
D = 128
DE = 16
B = 64
H = 2
L = 2

NC = 2
NS = 16
CHUNK = 128
ROWS = 2560
E_PAD = ROWS * CHUNK
NW = NC * NS
RPW = ROWS // NW
CH64 = 64
NT = 10240
NTAB = 10016
STRIPE = NT // NS

_MESH = plsc.VectorSubcoreMesh(core_axis_name="c", subcore_axis_name="s",
                               num_cores=NC, num_subcores=NS)


def _leaky(x, slope):
    return jnp.where(x >= 0, x, slope * x)


def _scmp_body(src_r, dst_r, edot_r, hdot_r, zdot_r, cv_r, z_r, e0_r,
               spart_out, aggp_out,
               hd_v, zd_v, srcb, dstb, edb, src64, dst64, ec, cv_v,
               zrows, erows, wrows, sem, s_sh, agg_sh):
    c = lax.axis_index("c")
    s = lax.axis_index("s")
    w = s * NC + c
    r0 = w * RPW

    pltpu.sync_copy(hdot_r, hd_v)
    pltpu.sync_copy(zdot_r, zd_v)
    pltpu.sync_copy(cv_r, cv_v)

    zero16 = jnp.zeros((16,), jnp.float32)

    for g in range(4):
        ec[pl.ds(g * 16, 16)] = zero16

    def _zw(k, _):
        for g in range(8):
            wrows[k, pl.ds(g * 16, 16)] = zero16
        return 0
    lax.fori_loop(0, CH64, _zw, 0)
    for t in range(STRIPE // 64):
        off = pl.multiple_of(s * STRIPE + t * 64, 64)
        pltpu.sync_copy(ec, s_sh.at[pl.ds(off, 64)])
        pltpu.sync_copy(wrows, agg_sh.at[pl.ds(off, 64)])
    plsc.subcore_barrier()

    cvv = cv_v[...]

    def _grp(gi, _):
        row = pl.multiple_of(r0 + gi * 8, 8)
        pltpu.sync_copy(src_r.at[pl.ds(row, 8)], srcb)
        pltpu.sync_copy(dst_r.at[pl.ds(row, 8)], dstb)
        pltpu.sync_copy(edot_r.at[pl.ds(row, 8)], edb)
        for jj in range(8):
            for hf in range(2):
                for g in range(4):
                    sl = pl.ds(g * 16, 16)
                    bsl = pl.ds(hf * 64 + g * 16, 16)
                    src64[sl] = srcb[jj, bsl]
                    dst64[sl] = dstb[jj, bsl]
                for g in range(4):
                    sl = pl.ds(g * 16, 16)
                    hv = plsc.load_gather(hd_v, [dst64[sl]])
                    zv = plsc.load_gather(zd_v, [src64[sl]])
                    ct = _leaky(hv + cvv, 0.2)
                    q = hv + zv + edb[jj, pl.ds(hf * 64 + g * 16, 16)]
                    ec[sl] = jnp.exp(_leaky(q, 0.2) - ct)
                pltpu.sync_copy(ec, s_sh.at[dst64], add=True)

                eoff = pl.multiple_of(
                    (r0 + gi * 8 + jj) * CHUNK + hf * 64, 64)
                pltpu.sync_copy(e0_r.at[pl.ds(eoff, 64)], erows)
                pltpu.async_copy(z_r.at[src64], zrows, sem).wait()

                def _k(k, _):
                    av = plsc.load_gather(ec, [jnp.full((16,), k, jnp.int32)])
                    for g in range(8):
                        sl = pl.ds(g * 16, 16)
                        wrows[k, sl] = (zrows[k, sl] + erows[k, sl]) * av
                    return 0
                lax.fori_loop(0, CH64, _k, 0)
                pltpu.sync_copy(wrows, agg_sh.at[dst64], add=True)
        return 0
    lax.fori_loop(0, RPW // 8, _grp, 0)
    plsc.subcore_barrier()

    for t in range(STRIPE // CH64):
        off = pl.multiple_of(s * STRIPE + t * CH64, CH64)
        pltpu.sync_copy(agg_sh.at[pl.ds(off, CH64)],
                        aggp_out.at[c, pl.ds(off, CH64)])

    @pl.when(s == 0)
    def _():
        pltpu.sync_copy(s_sh, spart_out.at[pl.ds(pl.multiple_of(c * NT, 8), NT)])


_scmp = functools.partial(
    pl.kernel, _scmp_body,
    out_type=(jax.ShapeDtypeStruct((NC * NT,), jnp.float32),
              jax.ShapeDtypeStruct((NC, NT, D), jnp.float32)),
    mesh=_MESH,
    compiler_params=pltpu.CompilerParams(needs_layout_passes=False),
    scratch_types=[
        pltpu.VMEM((NTAB,), jnp.float32),
        pltpu.VMEM((NTAB,), jnp.float32),
        pltpu.VMEM((8, CHUNK), jnp.int32),
        pltpu.VMEM((8, CHUNK), jnp.int32),
        pltpu.VMEM((8, CHUNK), jnp.float32),
        pltpu.VMEM((CH64,), jnp.int32),
        pltpu.VMEM((CH64,), jnp.int32),
        pltpu.VMEM((CH64,), jnp.float32),
        pltpu.VMEM((16,), jnp.float32),
        pltpu.VMEM((CH64, D), jnp.float32),
        pltpu.VMEM((CH64, D), jnp.float32),
        pltpu.VMEM((CH64, D), jnp.float32),
        pltpu.SemaphoreType.DMA,
        pltpu.VMEM_SHARED((NT,), jnp.float32),
        pltpu.VMEM_SHARED((NT, D), jnp.float32),
    ],
)()


def _bn_sums(y, n):
    mu = y.sum(0) / n
    var = (y * y).sum(0) / n - mu * mu
    return (y - mu) / jnp.sqrt(var + 1e-5)


def kernel(x, edge_index, edge_attr, batch,
           Wn_o, bn_o, We_o, be_o, Wm_o, a_o,
           Wn_f, bn_f, We_f, be_f, Wm_f, a_f,
           Wn_j, bn_j, We_j, be_j, Wm_j, a_j,
           Wo_o, bo_o, Wo_f, bo_f,
           Wp1, bp1, Wp2a, bp2a, Wp2b, bp2b):
    src = edge_index[0]
    dst = edge_index[1]
    src2 = jnp.concatenate(
        [src, jnp.zeros((E_PAD - E,), jnp.int32)]).reshape(ROWS, CHUNK)
    dst2 = jnp.concatenate(
        [dst, jnp.full((E_PAD - E,), N, jnp.int32)]).reshape(ROWS, CHUNK)

    params = {
        "o": (Wn_o, bn_o, We_o, be_o, Wm_o, a_o),
        "f": (Wn_f, bn_f, We_f, be_f, Wm_f, a_f),
        "j": (Wn_j, bn_j, We_j, be_j, Wm_j, a_j),
    }
    outs = {}
    for ch in ["o", "f", "j"]:
        Wn, bnb, We, be, Wm, a = params[ch]
        h0 = jax.nn.leaky_relu(_bn_sums(x @ Wn + bnb, N))
        e0 = jax.nn.leaky_relu(_bn_sums(edge_attr @ We + be, E))
        e0p = jnp.concatenate([e0, jnp.zeros((E_PAD - E, D), jnp.float32)])
        A2 = a[:, :, D:].reshape(H * L, D).T
        edots = e0 @ A2
        emax = edots.max(axis=0)
        heads = []
        for hi in range(H):
            h = h0
            for li in range(L):
                z = h @ Wm[hi, li]
                hdot = h @ a[hi, li, :D]
                zdot = z @ a[hi, li, D:]
                k = hi * L + li
                edot2 = jnp.concatenate(
                    [edots[:, k],
                     jnp.zeros((E_PAD - E,), jnp.float32)]).reshape(ROWS, CHUNK)
                cvec = jnp.full((16,), zdot.max() + emax[k], jnp.float32)
                hdot_t = jnp.concatenate(
                    [hdot, jnp.zeros((NTAB - N,), jnp.float32)])
                zdot_t = jnp.concatenate(
                    [zdot, jnp.zeros((NTAB - N,), jnp.float32)])
                spart, aggp = _scmp(src2, dst2, edot2, hdot_t, zdot_t, cvec,
                                    z, e0p)
                sden = spart.reshape(NC, NT).sum(0)
                agg = aggp[0, :N] + aggp[1, :N]
                agg = agg / jnp.maximum(sden[:N], 1e-30)[:, None]
                h = jax.nn.elu(agg) + h
            heads.append(jax.ops.segment_sum(h, batch, num_segments=B))
        if ch == "j":
            outs[ch] = jax.nn.relu(jnp.mean(jnp.stack(heads, axis=1), axis=1))
        else:
            cat = jnp.concatenate(heads, axis=-1)
            Wo, bo = (Wo_o, bo_o) if ch == "o" else (Wo_f, bo_f)
            outs[ch] = jax.nn.relu(_bn_sums(cat @ Wo + bo, B))
    zc = jnp.concatenate([outs["o"], outs["f"], outs["j"]], axis=-1)
    zc = _bn_sums(_leaky(zc @ Wp1 + bp1, 1e-7), B)
    zc = _leaky(zc @ Wp2a + bp2a, 1e-7)
    zc = _leaky(zc @ Wp2b + bp2b, 1e-7)
    return zc


# speedup vs baseline: 4.5763x; 1.0153x over previous
import functools

import jax
import jax.numpy as jnp
from jax import lax
from jax.experimental import pallas as pl
from jax.experimental.pallas import tpu as pltpu
from jax.experimental.pallas import tpu_sc as plsc

N = 10000
E = 320000
D = 128
DE = 16
B = 64
H = 2
L = 2

NC = 2
NS = 16
CHUNK = 128
ROWS = 2560
E_PAD = ROWS * CHUNK
NW = NC * NS
RPW = ROWS // NW
CH64 = 64
NT = 10240
NTAB = 10112
STRIPE = NT // NS

def _split(a):
    au = jax.lax.bitcast_convert_type(a, jnp.uint32)
    ah = jax.lax.bitcast_convert_type(au & jnp.uint32(0xFFFF0000),
                                      jnp.float32)
    al = a - ah
    return ah.astype(jnp.bfloat16), al.astype(jnp.bfloat16)


def _dot(a, b):
    ah, al = _split(a)
    bh, bl = _split(b)

    def f(u, v):
        return jnp.dot(u, v, preferred_element_type=jnp.float32)
    return f(ah, bh) + f(ah, bl) + f(al, bh)


def _leaky(x, slope):
    return jnp.where(x >= 0, x, slope * x)


def _scmp_body(src_r, dst_r, edot_r, hdot_r, zdot_r, cv_r, z_r, e0_r,
               spart_out, aggp_out,
               hd_v, zd_v, srcb, dstb, edb, src64, dst64, ec, cv_v,
               zrows, erows, wrows, sem, s_sh, agg_sh):
    c = lax.axis_index("c")
    s = lax.axis_index("s")
    w = s * NC + c
    r0 = w * RPW

    pltpu.sync_copy(hdot_r, hd_v)
    pltpu.sync_copy(zdot_r, zd_v)
    pltpu.sync_copy(cv_r, cv_v)

    zero16 = jnp.zeros((16,), jnp.float32)

    for g in range(4):
        ec[pl.ds(g * 16, 16)] = zero16

    def _zw(k, _):
        for g in range(8):
            wrows[k, pl.ds(g * 16, 16)] = zero16
        return 0
    lax.fori_loop(0, CH64, _zw, 0)
    for t in range(STRIPE // 64):
        off = pl.multiple_of(s * STRIPE + t * 64, 64)
        pltpu.sync_copy(ec, s_sh.at[pl.ds(off, 64)])
        pltpu.sync_copy(wrows, agg_sh.at[pl.ds(off, 64)])
    plsc.subcore_barrier()

    cvv = cv_v[...]

    def _grp(gi, _):
        row = pl.multiple_of(r0 + gi * 8, 8)
        pltpu.sync_copy(src_r.at[pl.ds(row, 8)], srcb)
        pltpu.sync_copy(dst_r.at[pl.ds(row, 8)], dstb)
        pltpu.sync_copy(edot_r.at[pl.ds(row, 8)], edb)
        for jj in range(8):
            for hf in range(2):
                for g in range(4):
                    sl = pl.ds(g * 16, 16)
                    bsl = pl.ds(hf * 64 + g * 16, 16)
                    src64[sl] = srcb[jj, bsl]
                    dst64[sl] = dstb[jj, bsl]
                for g in range(4):
                    sl = pl.ds(g * 16, 16)
                    hv = plsc.load_gather(hd_v, [dst64[sl]])
                    zv = plsc.load_gather(zd_v, [src64[sl]])
                    ct = _leaky(hv + cvv, 0.2)
                    q = hv + zv + edb[jj, pl.ds(hf * 64 + g * 16, 16)]
                    ec[sl] = jnp.exp(_leaky(q, 0.2) - ct + 66.54212933375474)
                pltpu.sync_copy(ec, s_sh.at[dst64], add=True)

                eoff = pl.multiple_of(
                    (r0 + gi * 8 + jj) * CHUNK + hf * 64, 64)
                pltpu.sync_copy(e0_r.at[pl.ds(eoff, 64)], erows)
                pltpu.async_copy(z_r.at[src64], zrows, sem).wait()

                def _k(k, _):
                    av = plsc.load_gather(ec, [jnp.full((16,), k, jnp.int32)])
                    for g in range(8):
                        sl = pl.ds(g * 16, 16)
                        wrows[k, sl] = (zrows[k, sl] + erows[k, sl]) * av
                    return 0
                lax.fori_loop(0, CH64, _k, 0)
                pltpu.sync_copy(wrows, agg_sh.at[dst64], add=True)
        return 0
    lax.fori_loop(0, RPW // 8, _grp, 0)
    plsc.subcore_barrier()

    for t in range(STRIPE // CH64):
        off = pl.multiple_of(s * STRIPE + t * CH64, CH64)
        pltpu.sync_copy(agg_sh.at[pl.ds(off, CH64)],
                        aggp_out.at[c, pl.ds(off, CH64)])

    @pl.when(s == 0)
    def _():
        pltpu.sync_copy(s_sh, spart_out.at[pl.ds(pl.multiple_of(c * NT, 8), NT)])


@functools.lru_cache(maxsize=None)
def _get_scmp():
  return functools.partial(
    pl.kernel, _scmp_body,
    out_type=(jax.ShapeDtypeStruct((NC * NT,), jnp.float32),
              jax.ShapeDtypeStruct((NC, NT, D), jnp.float32)),
    mesh=plsc.VectorSubcoreMesh(core_axis_name="c", subcore_axis_name="s",
                                num_cores=NC, num_subcores=NS),
    compiler_params=pltpu.CompilerParams(needs_layout_passes=False),
    scratch_types=[
        pltpu.VMEM((NTAB,), jnp.float32),
        pltpu.VMEM((NTAB,), jnp.float32),
        pltpu.VMEM((8, CHUNK), jnp.int32),
        pltpu.VMEM((8, CHUNK), jnp.int32),
        pltpu.VMEM((8, CHUNK), jnp.float32),
        pltpu.VMEM((CH64,), jnp.int32),
        pltpu.VMEM((CH64,), jnp.int32),
        pltpu.VMEM((CH64,), jnp.float32),
        pltpu.VMEM((16,), jnp.float32),
        pltpu.VMEM((CH64, D), jnp.float32),
        pltpu.VMEM((CH64, D), jnp.float32),
        pltpu.VMEM((CH64, D), jnp.float32),
        pltpu.SemaphoreType.DMA,
        pltpu.VMEM_SHARED((NT,), jnp.float32),
        pltpu.VMEM_SHARED((NT, D), jnp.float32),
    ],
  )()


def _proh_body(x_ref, Wn_ref, bn_ref, h0_ref):
    y = _dot(x_ref[...], Wn_ref[0]) + bn_ref[0]
    mu = jnp.mean(y, axis=0)
    var = jnp.mean(y * y, axis=0) - mu * mu
    h0_ref[0] = _leaky((y - mu) / jnp.sqrt(var + 1e-5), 0.01)


_k_proh = pl.pallas_call(
    _proh_body,
    grid=(3,),
    in_specs=[
        pl.BlockSpec((N, D), lambda c: (0, 0)),
        pl.BlockSpec((1, D, D), lambda c: (c, 0, 0)),
        pl.BlockSpec((1, 1, D), lambda c: (c, 0, 0)),
    ],
    out_specs=pl.BlockSpec((1, N, D), lambda c: (c, 0, 0)),
    out_shape=jax.ShapeDtypeStruct((3, N, D), jnp.float32),
)

EB1 = 16000


def _gram_body(ea_ref, S_ref, m_ref):
    @pl.when(pl.program_id(0) == 0)
    def _():
        S_ref[...] = jnp.zeros((DE, DE), jnp.float32)
        m_ref[...] = jnp.zeros((1, DE), jnp.float32)
    ea = ea_ref[...]
    eah, eal = _split(ea)

    def g(u, v):
        return lax.dot_general(u, v, (((0,), (0,)), ((), ())),
                               preferred_element_type=jnp.float32)
    S_ref[...] += g(eah, eah) + g(eah, eal) + g(eal, eah)
    m_ref[...] += jnp.sum(ea, axis=0)[None, :]


_k_gram = pl.pallas_call(
    _gram_body,
    grid=(E // EB1,),
    in_specs=[pl.BlockSpec((EB1, DE), lambda i: (i, 0))],
    out_specs=[pl.BlockSpec((DE, DE), lambda i: (0, 0)),
               pl.BlockSpec((1, DE), lambda i: (0, 0))],
    out_shape=[jax.ShapeDtypeStruct((DE, DE), jnp.float32),
               jax.ShapeDtypeStruct((1, DE), jnp.float32)],
)

EB2 = 8192


def _e0_body(ea_ref, We_ref, be_ref, S_ref, m_ref, A2_ref,
             e0_ref, ed_ref, pm_ref):
    We = We_ref[...]
    be = be_ref[...]
    mu = _dot(m_ref[...] / E, We) + be
    T = _dot(S_ref[...], We)
    diag = jnp.sum(We * T, axis=0)
    var = diag / E - (mu[0] - be[0]) ** 2
    inv = 1.0 / jnp.sqrt(var + 1e-5)
    y = _dot(ea_ref[...], We) + be
    e0 = _leaky((y - mu) * inv[None, :], 0.01)
    e0_ref[...] = e0
    ed = _dot(e0, A2_ref[...])
    ed_ref[...] = ed

    @pl.when(pl.program_id(0) == 0)
    def _():
        pm_ref[...] = jnp.full((1, 8), -jnp.inf, jnp.float32)
    pm_ref[...] = jnp.maximum(pm_ref[...], jnp.max(ed, axis=0)[None, :])


_k_e0 = pl.pallas_call(
    _e0_body,
    grid=(E_PAD // EB2,),
    in_specs=[
        pl.BlockSpec((EB2, DE), lambda i: (i, 0)),
        pl.BlockSpec((DE, D), lambda i: (0, 0)),
        pl.BlockSpec((1, D), lambda i: (0, 0)),
        pl.BlockSpec((DE, DE), lambda i: (0, 0)),
        pl.BlockSpec((1, DE), lambda i: (0, 0)),
        pl.BlockSpec((D, 8), lambda i: (0, 0)),
    ],
    out_specs=[pl.BlockSpec((EB2, D), lambda i: (i, 0)),
               pl.BlockSpec((EB2, 8), lambda i: (i, 0)),
               pl.BlockSpec((1, 8), lambda i: (0, 0))],
    out_shape=[jax.ShapeDtypeStruct((E_PAD, D), jnp.float32),
               jax.ShapeDtypeStruct((E_PAD, 8), jnp.float32),
               jax.ShapeDtypeStruct((1, 8), jnp.float32)],
)


def _step_body(has_agg, *refs):
    if has_agg:
        (h_ref, Wm_ref, a1_ref, a2_ref, me_ref, aggp_ref, sp_ref,
         hn_ref, z_ref, hd_ref, zd_ref, cv_ref) = refs
        sden = sp_ref[0, :, 0] + sp_ref[1, :, 0]
        sden = jnp.maximum(sden, 1e-30)
        agg = (aggp_ref[0] + aggp_ref[1]) / sden[:, None]
        hh = jnp.where(agg > 0, agg, jnp.exp(agg) - 1.0) + h_ref[...]
    else:
        (h_ref, Wm_ref, a1_ref, a2_ref, me_ref,
         hn_ref, z_ref, hd_ref, zd_ref, cv_ref) = refs
        hh = h_ref[...]
    hn_ref[...] = hh
    z = _dot(hh, Wm_ref[...])
    z_ref[...] = z
    hd_ref[...] = _dot(hh, a1_ref[...])
    zd = _dot(z, a2_ref[...])
    zd_ref[...] = zd

    @pl.when(pl.program_id(0) == 0)
    def _():
        cv_ref[...] = me_ref[...]
    cv_ref[...] = jnp.maximum(cv_ref[...], jnp.max(zd) + me_ref[...])


NB = 2000


def _mk_step(has_agg):
    in_specs = [
        pl.BlockSpec((NB, D), lambda i: (i, 0)),
        pl.BlockSpec((D, D), lambda i: (0, 0)),
        pl.BlockSpec((D, 1), lambda i: (0, 0)),
        pl.BlockSpec((D, 1), lambda i: (0, 0)),
        pl.BlockSpec((1, 16), lambda i: (0, 0)),
    ]
    if has_agg:
        in_specs += [
            pl.BlockSpec((NC, NB, D), lambda i: (0, i, 0)),
            pl.BlockSpec((NC, NB, 1), lambda i: (0, i, 0)),
        ]
    return pl.pallas_call(
        functools.partial(_step_body, has_agg),
        grid=(N // NB,),
        in_specs=in_specs,
        out_specs=[pl.BlockSpec((NB, D), lambda i: (i, 0)),
                   pl.BlockSpec((NB, D), lambda i: (i, 0)),
                   pl.BlockSpec((NB, 1), lambda i: (i, 0)),
                   pl.BlockSpec((NB, 1), lambda i: (i, 0)),
                   pl.BlockSpec((1, 16), lambda i: (0, 0))],
        out_shape=[jax.ShapeDtypeStruct((N, D), jnp.float32),
                   jax.ShapeDtypeStruct((N, D), jnp.float32),
                   jax.ShapeDtypeStruct((N, 1), jnp.float32),
                   jax.ShapeDtypeStruct((N, 1), jnp.float32),
                   jax.ShapeDtypeStruct((1, 16), jnp.float32)],
    )


_k_step0 = _mk_step(False)
_k_step1 = _mk_step(True)


def _rou_body(h_ref, aggp_ref, sp_ref, batch_ref, ro_ref, hh_sc):
    sden = sp_ref[0, :, 0] + sp_ref[1, :, 0]
    sden = jnp.maximum(sden, 1e-30)
    agg = (aggp_ref[0] + aggp_ref[1]) / sden[:, None]
    hh_sc[...] = jnp.where(agg > 0, agg, jnp.exp(agg) - 1.0) + h_ref[...]

    @pl.when(pl.program_id(0) == 0)
    def _():
        ro_ref[...] = jnp.zeros((B, D), jnp.float32)

    acc = jnp.zeros((B, D), jnp.float32)
    for t in range(8):
        sl = pl.ds(t * 250, 250)
        bb = batch_ref[sl, 0]
        seg = lax.broadcasted_iota(jnp.int32, (B, 250), 0)
        oh = (bb[None, :] == seg).astype(jnp.bfloat16)
        xh, xl = _split(hh_sc[sl, :])
        acc += (jnp.dot(oh, xh, preferred_element_type=jnp.float32)
                + jnp.dot(oh, xl, preferred_element_type=jnp.float32))
    ro_ref[...] += acc


_k_rou = pl.pallas_call(
    _rou_body,
    grid=(N // NB,),
    in_specs=[
        pl.BlockSpec((NB, D), lambda i: (i, 0)),
        pl.BlockSpec((NC, NB, D), lambda i: (0, i, 0)),
        pl.BlockSpec((NC, NB, 1), lambda i: (0, i, 0)),
        pl.BlockSpec((NB, 1), lambda i: (i, 0)),
    ],
    out_specs=pl.BlockSpec((B, D), lambda i: (0, 0)),
    out_shape=jax.ShapeDtypeStruct((B, D), jnp.float32),
    scratch_shapes=[pltpu.VMEM((NB, D), jnp.float32)],
)


def _tail_body(ro_ref, Woo_ref, boo_ref, Wof_ref, bof_ref,
               Wp1_ref, bp1_ref, W2a_ref, b2a_ref, W2b_ref, b2b_ref,
               out_ref):
    def bn(y):
        mu = jnp.mean(y, axis=0)
        var = jnp.mean(y * y, axis=0) - mu * mu
        return (y - mu) / jnp.sqrt(var + 1e-5)

    ro = ro_ref[...]
    co = _dot(jnp.concatenate([ro[0], ro[1]], axis=1), Woo_ref[...]) + boo_ref[...]
    o = jnp.maximum(bn(co), 0.0)
    cf = _dot(jnp.concatenate([ro[2], ro[3]], axis=1), Wof_ref[...]) + bof_ref[...]
    f = jnp.maximum(bn(cf), 0.0)
    jj = jnp.maximum((ro[4] + ro[5]) * 0.5, 0.0)
    zc = _dot(jnp.concatenate([o, f, jj], axis=1), Wp1_ref[...]) + bp1_ref[...]
    zc = bn(_leaky(zc, 1e-7))
    zc = _leaky(_dot(zc, W2a_ref[...]) + b2a_ref[...], 1e-7)
    zc = _leaky(_dot(zc, W2b_ref[...]) + b2b_ref[...], 1e-7)
    out_ref[...] = zc


_k_tail = pl.pallas_call(
    _tail_body,
    out_shape=jax.ShapeDtypeStruct((B, 1), jnp.float32),
)


def _bn_sums(y, n):
    mu = y.sum(0) / n
    var = (y * y).sum(0) / n - mu * mu
    return (y - mu) / jnp.sqrt(var + 1e-5)


def kernel(x, edge_index, edge_attr, batch,
           Wn_o, bn_o, We_o, be_o, Wm_o, a_o,
           Wn_f, bn_f, We_f, be_f, Wm_f, a_f,
           Wn_j, bn_j, We_j, be_j, Wm_j, a_j,
           Wo_o, bo_o, Wo_f, bo_f,
           Wp1, bp1, Wp2a, bp2a, Wp2b, bp2b):
    src = edge_index[0]
    dst = edge_index[1]
    src2 = jnp.concatenate(
        [src, jnp.zeros((E_PAD - E,), jnp.int32)]).reshape(ROWS, CHUNK)
    dst2 = jnp.concatenate(
        [dst, jnp.full((E_PAD - E,), N, jnp.int32)]).reshape(ROWS, CHUNK)
    eap = jnp.concatenate(
        [edge_attr, jnp.zeros((E_PAD - E, DE), jnp.float32)])

    def _bnp(y):
        mu = y.mean(0)
        var = (y * y).mean(0) - mu * mu
        return (y - mu) / jnp.sqrt(var + 1e-5)
    h0_all = jnp.stack([
        jax.nn.leaky_relu(_bnp(x @ W + bb))
        for W, bb in [(Wn_o, bn_o), (Wn_f, bn_f), (Wn_j, bn_j)]])
    S, m = _k_gram(edge_attr)

    params = {"o": (Wm_o, a_o), "f": (Wm_f, a_f), "j": (Wm_j, a_j)}
    Wes = {"o": (We_o, be_o), "f": (We_f, be_f), "j": (We_j, be_j)}
    ros = []
    for ci, ch in enumerate(["o", "f", "j"]):
        Wm, a = params[ch]
        We, be = Wes[ch]
        A2 = a[:, :, D:].reshape(H * L, D).T
        A2p = jnp.concatenate(
            [A2, jnp.zeros((D, 8 - H * L), jnp.float32)], axis=1)
        ye = edge_attr @ We + be
        mu = ye.mean(0)
        var = (ye * ye).mean(0) - mu * mu
        e0j = jax.nn.leaky_relu((ye - mu) / jnp.sqrt(var + 1e-5))
        e0p = jnp.concatenate([e0j, jnp.zeros((E_PAD - E, D), jnp.float32)])
        edj = e0j @ A2
        ed = jnp.concatenate([edj, jnp.zeros((E_PAD - E, H * L), jnp.float32)])
        pm = edj.max(0)[None, :]
        h0 = h0_all[ci]
        for hi in range(H):
            h = h0
            spart = aggp = None
            for li in range(L):
                k = hi * L + li
                if li == 0:
                    hh = h
                else:
                    sden0 = jnp.maximum(
                        spart.reshape(NC, NT)[:, :N].sum(0), 1e-30)
                    agg0 = (aggp[0, :N] + aggp[1, :N]) / sden0[:, None]
                    hh = jax.nn.elu(agg0) + h
                z = hh @ Wm[hi, li]
                hd = (hh @ a[hi, li, :D])[:, None]
                zd = (z @ a[hi, li, D:])[:, None]
                cv = jnp.full((1, 16), jnp.max(zd) + pm[0, k], jnp.float32)
                edot2 = ed[:, k].reshape(ROWS, CHUNK)
                hdot_t = jnp.concatenate(
                    [hd[:, 0], jnp.zeros((NTAB - N,), jnp.float32)])
                zdot_t = jnp.concatenate(
                    [zd[:, 0], jnp.zeros((NTAB - N,), jnp.float32)])
                spart, aggp = _get_scmp()(src2, dst2, edot2, hdot_t, zdot_t,
                                          cv.reshape(16), z, e0p)
                h = hh
            sden = jnp.maximum(spart.reshape(NC, NT)[:, :N].sum(0), 1e-30)
            agg = (aggp[0, :N] + aggp[1, :N]) / sden[:, None]
            hfin = jax.nn.elu(agg) + h
            ros.append(jax.ops.segment_sum(hfin, batch, num_segments=B))
    def _bn(y):
        mu = y.mean(0)
        var = (y * y).mean(0) - mu * mu
        return (y - mu) / jnp.sqrt(var + 1e-5)
    o = jax.nn.relu(_bn(jnp.concatenate([ros[0], ros[1]], 1) @ Wo_o + bo_o))
    f = jax.nn.relu(_bn(jnp.concatenate([ros[2], ros[3]], 1) @ Wo_f + bo_f))
    jj = jax.nn.relu((ros[4] + ros[5]) * 0.5)
    zc = _bn(jax.nn.leaky_relu(jnp.concatenate([o, f, jj], 1) @ Wp1 + bp1, 1e-7))
    zc = jax.nn.leaky_relu(zc @ Wp2a + bp2a, 1e-7)
    zc = jax.nn.leaky_relu(zc @ Wp2b + bp2b, 1e-7)
    return zc
